# baseline (speedup 1.0000x reference)
import functools

import numpy as np
import jax
import jax.numpy as jnp
from jax import lax
from jax.experimental import pallas as pl
from jax.experimental.pallas import tpu as pltpu
from jax.experimental.pallas import tpu_sc as plsc

MAXF = 1024
D = 256
NF = 10
DOUT = D + 2 * NF
DPAD = 384
B = 16384
F = 3
DW = 2 * DOUT

NC = 2
NS = 16
L = 16
NW = NC * NS

BPW = B // NW
RC = 32
NCH = BPW // RC
NFULL = DOUT // L
REM = DOUT - NFULL * L


def _fourier_lut() -> np.ndarray:
    time_resolution = 2.0 ** np.ceil(np.log2(MAXF))
    powers = 2.0 ** np.arange(NF)
    coefs = (powers * np.pi / time_resolution).astype(np.float32)
    d = np.arange(MAXF, dtype=np.float32)
    raw = (d[:, None] * coefs[None, :]).astype(np.float32)
    lut = np.zeros((MAXF, DPAD - D), dtype=np.float32)
    lut[:, :NF] = np.sin(raw.astype(np.float64)).astype(np.float32)
    lut[:, NF:2 * NF] = np.cos(raw.astype(np.float64)).astype(np.float32)
    return lut


_LUT = _fourier_lut()


@functools.partial(
    pl.kernel,
    out_type=jax.ShapeDtypeStruct((B, DW), jnp.float32),
    mesh=plsc.VectorSubcoreMesh(core_axis_name="c", subcore_axis_name="s",
                                num_cores=NC, num_subcores=NS),
    scratch_types=[
        pltpu.VMEM((BPW * F,), jnp.int32),
        pltpu.VMEM((NCH, RC), jnp.int32),
        pltpu.VMEM((NCH, RC), jnp.int32),
        pltpu.VMEM((3, RC, DW), jnp.float32),
        pltpu.VMEM((3, RC, DPAD), jnp.float32),
        pltpu.SemaphoreType.DMA,
        pltpu.SemaphoreType.DMA,
        pltpu.SemaphoreType.DMA,
        pltpu.SemaphoreType.DMA,
        pltpu.SemaphoreType.DMA,
        pltpu.SemaphoreType.DMA,
        pltpu.SemaphoreType.DMA,
        pltpu.SemaphoreType.DMA,
        pltpu.SemaphoreType.DMA,
    ],
    compiler_params=pltpu.CompilerParams(
        needs_layout_passes=False, use_tc_tiling_on_sc=True),
)
def _sc_encode(t_hbm, tab_hbm, out_hbm, t_v, idxe_v, idxo_v, abuf_v, obuf_v,
               ge0, ge1, ge2, go0, go1, go2, os0, os1, os2):
    wid = lax.axis_index("s") * NC + lax.axis_index("c")
    pltpu.sync_copy(t_hbm.at[pl.ds(wid * (BPW * F), BPW * F)], t_v)

    lanes = jnp.arange(L, dtype=jnp.int32)
    for g in range(BPW // L):
        pos = (lanes + g * L) * 3
        t0 = plsc.load_gather(t_v, [pos])
        t1 = plsc.load_gather(t_v, [pos + 1])
        t2 = plsc.load_gather(t_v, [pos + 2])
        c, col = divmod(g * L, RC)
        idxe_v[c, pl.ds(col, L)] = t1 - t0
        idxo_v[c, pl.ds(col, L)] = t2 - t1

    gsems = ((ge0, go0), (ge1, go1), (ge2, go2))
    osems = (os0, os1, os2)

    def start_gathers(c, s):
        ge = pltpu.async_copy(tab_hbm.at[idxe_v.at[c]],
                              abuf_v.at[s, :, pl.ds(0, DPAD)], gsems[s][0])
        go = pltpu.async_copy(tab_hbm.at[idxo_v.at[c]],
                              obuf_v.at[s], gsems[s][1])
        return ge, go

    cross = {g for g in range(NFULL) if (DOUT + L * g) % 128 > 128 - L}
    rem_m = lanes < REM
    rem_src = (NFULL * L) + jnp.where(rem_m, lanes, 0)
    rem_dst = rem_src + DOUT

    def assemble(s):
        def row(i, _):
            rows16 = jnp.full((L,), i, jnp.int32)
            for g in range(NFULL):
                x = obuf_v[s, i, pl.ds(L * g, L)]
                if g in cross:
                    plsc.store_scatter(
                        abuf_v.at[s], [rows16, lanes + (DOUT + L * g)], x)
                else:
                    abuf_v[s, i, pl.ds(DOUT + L * g, L)] = x
            x = plsc.load_gather(obuf_v.at[s], [rows16, rem_src], mask=rem_m)
            plsc.store_scatter(abuf_v.at[s], [rows16, rem_dst], x, mask=rem_m)
            return ()
        lax.fori_loop(0, RC, row, (), unroll=1)

    obase = wid * BPW
    gd = [None, None, None]
    od = [None, None, None]
    gd[0] = start_gathers(0, 0)
    gd[1] = start_gathers(1, 1)
    for c in range(NCH):
        s = c % 3
        gd[s][0].wait()
        gd[s][1].wait()
        assemble(s)
        od[s] = pltpu.async_copy(
            abuf_v.at[s], out_hbm.at[pl.ds(obase + c * RC, RC)], osems[s])
        if c + 2 < NCH:
            ns = (c + 2) % 3
            if od[ns] is not None:
                od[ns].wait()
            gd[ns] = start_gathers(c + 2, ns)
    od[(NCH - 2) % 3].wait()
    od[(NCH - 1) % 3].wait()


def kernel(t, embed_table):
    fused = jnp.concatenate(
        [embed_table, jnp.asarray(_LUT, dtype=jnp.float32)], axis=1)
    return _sc_encode(t.astype(jnp.int32).reshape(-1), fused)


# speedup vs baseline: 1.0282x; 1.0282x over previous
import functools

import numpy as np
import jax
import jax.numpy as jnp
from jax import lax
from jax.experimental import pallas as pl
from jax.experimental.pallas import tpu as pltpu
from jax.experimental.pallas import tpu_sc as plsc

MAXF = 1024
D = 256
NF = 10
DOUT = D + 2 * NF
B = 16384
F = 3
DW = 2 * DOUT
WHEEL = 2048
QUARTER = WHEEL // 4

NC = 2
NS = 16
L = 16
NW = NC * NS

RC = 32
NEMB = D // L


def _sine_wheel() -> np.ndarray:
    k = np.arange(WHEEL, dtype=np.float64)
    return np.sin(2.0 * np.pi * k / WHEEL).astype(np.float32)


_WHEEL = _sine_wheel()


def _make_band_encoder(rows):
    bpw = rows // NW
    nch = bpw // RC

    @functools.partial(
        pl.kernel,
        out_type=jax.ShapeDtypeStruct((rows, DW), jnp.float32),
        mesh=plsc.VectorSubcoreMesh(core_axis_name="c", subcore_axis_name="s",
                                    num_cores=NC, num_subcores=NS),
        scratch_types=[
            pltpu.VMEM((F, bpw), jnp.int32),
            pltpu.VMEM((WHEEL,), jnp.float32),
            pltpu.VMEM((nch, RC), jnp.int32),
            pltpu.VMEM((nch, RC), jnp.int32),
            pltpu.VMEM((3, RC, DW), jnp.float32),
            pltpu.VMEM((3, RC, D), jnp.float32),
            pltpu.SemaphoreType.DMA,
            pltpu.SemaphoreType.DMA,
            pltpu.SemaphoreType.DMA,
            pltpu.SemaphoreType.DMA,
            pltpu.SemaphoreType.DMA,
            pltpu.SemaphoreType.DMA,
            pltpu.SemaphoreType.DMA,
            pltpu.SemaphoreType.DMA,
            pltpu.SemaphoreType.DMA,
        ],
        compiler_params=pltpu.CompilerParams(
            needs_layout_passes=False, use_tc_tiling_on_sc=True),
    )
    def band(t_hbm, wheel_hbm, tab_hbm, out_hbm, t_v, wheel_v, idxe_v, idxo_v,
             abuf_v, obuf_v, ge0, ge1, ge2, go0, go1, go2, os0, os1, os2):
        wid = lax.axis_index("s") * NC + lax.axis_index("c")
        pltpu.sync_copy(t_hbm.at[:, pl.ds(wid * bpw, bpw)], t_v)
        pltpu.sync_copy(wheel_hbm, wheel_v)

        lanes = jnp.arange(L, dtype=jnp.int32)
        for g in range(bpw // L):
            t0 = t_v[0, pl.ds(g * L, L)]
            t1 = t_v[1, pl.ds(g * L, L)]
            t2 = t_v[2, pl.ds(g * L, L)]
            c, col = divmod(g * L, RC)
            idxe_v[c, pl.ds(col, L)] = t1 - t0
            idxo_v[c, pl.ds(col, L)] = t2 - t1

        gsems = ((ge0, go0), (ge1, go1), (ge2, go2))
        osems = (os0, os1, os2)

        def start_gathers(c, s):
            ge = pltpu.async_copy(tab_hbm.at[idxe_v.at[c]],
                                  abuf_v.at[s, :, pl.ds(0, D)],
                                  gsems[s][0])
            go = pltpu.async_copy(tab_hbm.at[idxo_v.at[c]],
                                  obuf_v.at[s], gsems[s][1])
            return ge, go

        def fourier(c, s):
            for half in range(2):
                rows16 = lanes + half * L
                de = idxe_v[c, pl.ds(half * L, L)]
                do = idxo_v[c, pl.ds(half * L, L)]
                for f in range(NF):
                    me = (de << f) & (WHEEL - 1)
                    mo = (do << f) & (WHEEL - 1)
                    se = plsc.load_gather(wheel_v, [me])
                    ce = plsc.load_gather(wheel_v, [(me + QUARTER)
                                                    & (WHEEL - 1)])
                    so = plsc.load_gather(wheel_v, [mo])
                    co = plsc.load_gather(wheel_v, [(mo + QUARTER)
                                                    & (WHEEL - 1)])
                    plsc.store_scatter(
                        abuf_v.at[s], [rows16, jnp.full((L,), D + f,
                                                        jnp.int32)], se)
                    plsc.store_scatter(
                        abuf_v.at[s], [rows16, jnp.full((L,), D + NF + f,
                                                        jnp.int32)], ce)
                    plsc.store_scatter(
                        abuf_v.at[s], [rows16, jnp.full((L,), DOUT + D + f,
                                                        jnp.int32)], so)
                    plsc.store_scatter(
                        abuf_v.at[s], [rows16, jnp.full((L,), DOUT + D + NF
                                                        + f, jnp.int32)], co)

        cross = {g for g in range(NEMB) if (DOUT + L * g) % 128 > 128 - L}

        def assemble(s):
            def row(i, _):
                rows16 = jnp.full((L,), i, jnp.int32)
                for g in range(NEMB):
                    x = obuf_v[s, i, pl.ds(L * g, L)]
                    if g in cross:
                        plsc.store_scatter(
                            abuf_v.at[s], [rows16, lanes + (DOUT + L * g)], x)
                    else:
                        abuf_v[s, i, pl.ds(DOUT + L * g, L)] = x
                return ()
            lax.fori_loop(0, RC, row, (), unroll=1)

        obase = wid * bpw
        gd = [None, None, None]
        od = [None, None, None]
        gd[0] = start_gathers(0, 0)
        gd[1] = start_gathers(1, 1)
        for c in range(nch):
            s = c % 3
            fourier(c, s)
            gd[s][0].wait()
            gd[s][1].wait()
            assemble(s)
            od[s] = pltpu.async_copy(
                abuf_v.at[s], out_hbm.at[pl.ds(obase + c * RC, RC)], osems[s])
            if c + 2 < nch:
                ns = (c + 2) % 3
                if od[ns] is not None:
                    od[ns].wait()
                gd[ns] = start_gathers(c + 2, ns)
        od[(nch - 2) % 3].wait()
        od[(nch - 1) % 3].wait()

    return band


NBANDS = 1
_band_encode = _make_band_encoder(B // NBANDS)


def kernel(t, embed_table):
    tt = t.astype(jnp.int32).T
    wheel = jnp.asarray(_WHEEL, dtype=jnp.float32)
    if NBANDS == 1:
        return _band_encode(tt, wheel, embed_table)
    br = B // NBANDS
    parts = [_band_encode(tt[:, i * br:(i + 1) * br], wheel, embed_table)
             for i in range(NBANDS)]
    return jnp.concatenate(parts, axis=0)
